# baseline (speedup 1.0000x reference)
import functools

import jax
import jax.numpy as jnp
import numpy as np
from jax import lax
from jax.experimental import pallas as pl
from jax.experimental.pallas import tpu as pltpu
from jax.experimental.pallas import tpu_sc as plsc

L, B, D = 200, 256, 64
K = 1000
N_ROWS = L * B
DP = 128

_SC_INFO = plsc.get_sparse_core_info()
_NC = _SC_INFO.num_cores
_NS = _SC_INFO.num_subcores
_NW = _NC * _NS
_CHUNK = N_ROWS // _NW
_HALF = _CHUNK // 2
_PIECES = [(o, min(128, _HALF - o)) for o in range(0, _HALF, 128)]

_BL = 50
_NSTEPS = L // _BL
_BLV = 20
_NSTEPS_V = (L + _BLV - 1) // _BLV


def _sc_gather_fn():
  mesh = plsc.VectorSubcoreMesh(core_axis_name="c", subcore_axis_name="s")

  @functools.partial(
      pl.kernel,
      mesh=mesh,
      compiler_params=pltpu.CompilerParams(use_tc_tiling_on_sc=False),
      out_type=(
          jax.ShapeDtypeStruct((N_ROWS, D), jnp.float32),
          jax.ShapeDtypeStruct((N_ROWS, D), jnp.float32),
      ),
      scratch_types=[
          pltpu.VMEM((_CHUNK,), jnp.int32),
          pltpu.VMEM((_HALF, DP), jnp.float32),
          pltpu.VMEM_SHARED((1000, DP), jnp.float32),
          pltpu.SemaphoreType.DMA,
      ],
  )
  def sc_gather(node_idx_h, edge_idx_h, node_tab_h, edge_tab_h,
                node_out_h, edge_out_h, idx_v, rows_v, etab_s, sem):
    wid = lax.axis_index("s") * _NC + lax.axis_index("c")
    base = wid * _CHUNK

    @pl.when(lax.axis_index("s") == 0)
    def _():
      pltpu.sync_copy(edge_tab_h, etab_s)

    def gather_to(tab, src_idx_h, out_h):
      pltpu.sync_copy(src_idx_h.at[pl.ds(base, _CHUNK)], idx_v)
      for half in range(2):
        hoff = half * _HALF
        handles = []
        for off, sz in _PIECES:
          handles.append(
              pltpu.async_copy(
                  tab.at[idx_v.at[pl.ds(hoff + off, sz)]],
                  rows_v.at[pl.ds(off, sz)],
                  sem,
              ))
        for h in handles:
          h.wait()
        pltpu.sync_copy(rows_v.at[:, pl.ds(0, D)],
                        out_h.at[pl.ds(base + hoff, _HALF)])

    gather_to(node_tab_h, node_idx_h, node_out_h)
    plsc.subcore_barrier()
    gather_to(etab_s, edge_idx_h, edge_out_h)

  return sc_gather


_sc_gather = _sc_gather_fn()


def _tc_val_body(vm_ref, vt_ref, pe_ref, out_ref):
  y_t = jax.lax.dot_general(
      vt_ref[...], vm_ref[...], (((1,), (0,)), ((), ())),
      preferred_element_type=jnp.float32)
  for j in range(_BLV):
    out_ref[0, j] = y_t[:, j * B:(j + 1) * B] + pe_ref[j, :, :1]


_tc_val = pl.pallas_call(
    _tc_val_body,
    grid=(_NSTEPS_V,),
    in_specs=[
        pl.BlockSpec((K, _BLV * B), lambda l: (0, l)),
        pl.BlockSpec((D, K), lambda l: (0, 0)),
        pl.BlockSpec((_BLV, D, 8), lambda l: (l, 0, 0)),
    ],
    out_specs=pl.BlockSpec((1, _BLV, D, B), lambda l: (2, l, 0, 0)),
    out_shape=jax.ShapeDtypeStruct((3, L, D, B), jnp.float32),
)


_NPAIR = B // 2


def _tc_asm_body(buf_ref, nv_ref, ev_ref, pe_ref, seso_ref, out_ref):
  del buf_ref
  seso = seso_ref[...]
  for part, ref in ((0, nv_ref), (1, ev_ref)):
    for j in range(_BL):
      pj = ref[pl.ds(j * _NPAIR, _NPAIR), :]
      lhs = jnp.concatenate([pj[:, :D], pj[:, D:]], axis=0)
      xt = jax.lax.dot_general(
          lhs, seso, (((0,), (0,)), ((), ())),
          preferred_element_type=jnp.float32)
      out_ref[part, j] = xt + pe_ref[j, :, :1]


_tc_asm = pl.pallas_call(
    _tc_asm_body,
    grid=(_NSTEPS,),
    in_specs=[
        pl.BlockSpec(memory_space=pltpu.MemorySpace.HBM),
        pl.BlockSpec((_BL * _NPAIR, DP), lambda l: (l, 0)),
        pl.BlockSpec((_BL * _NPAIR, DP), lambda l: (l, 0)),
        pl.BlockSpec((_BL, D, 8), lambda l: (l, 0, 0)),
        pl.BlockSpec((B, B), lambda l: (0, 0)),
    ],
    out_specs=pl.BlockSpec((2, _BL, D, B), lambda l: (0, l, 0, 0)),
    out_shape=jax.ShapeDtypeStruct((3, L, D, B), jnp.float32),
    input_output_aliases={0: 0},
)


def _pad_body(in_ref, out_ref):
  xt = in_ref[...].T
  out_ref[...] = jnp.concatenate([xt, jnp.zeros_like(xt)], axis=1)


def _make_pad(n_rows, block_rows):
  return pl.pallas_call(
      _pad_body,
      grid=((n_rows + block_rows - 1) // block_rows,),
      in_specs=[pl.BlockSpec((D, block_rows), lambda i: (0, i))],
      out_specs=pl.BlockSpec((block_rows, DP), lambda i: (i, 0)),
      out_shape=jax.ShapeDtypeStruct((n_rows, DP), jnp.float32),
  )


_pad_node = _make_pad(100000, 16384)
_pad_edge = _make_pad(1000, 1000)


def _pos_encoding_np():
  pos = np.arange(L, dtype=np.float32)[:, None]
  div = np.exp(np.arange(0, D, 2, dtype=np.float32) * (-np.log(10000.0) / D))
  pe = np.zeros((L, D), dtype=np.float32)
  pe[:, 0::2] = np.sin(pos * div).astype(np.float32)
  pe[:, 1::2] = np.cos(pos * div).astype(np.float32)
  return pe


_PE_MINI = np.broadcast_to(_pos_encoding_np()[:, :, None], (L, D, 8)).copy()
_SESO = np.zeros((B, B), dtype=np.float32)
_SESO[np.arange(B // 2), 2 * np.arange(B // 2)] = 1.0
_SESO[B // 2 + np.arange(B // 2), 2 * np.arange(B // 2) + 1] = 1.0


def kernel(node_idx, edge_idx, node_val_mat, node_embed_table,
           edge_embed_table, val_tok_embed):
  pe_mini = jnp.asarray(_PE_MINI)
  seso = jnp.asarray(_SESO)
  node_tab_p = _pad_node(node_embed_table.T)
  edge_tab_p = _pad_edge(edge_embed_table.T)
  node_rows, edge_rows = _sc_gather(
      node_idx.reshape(-1), edge_idx.reshape(-1), node_tab_p, edge_tab_p)
  val_out = _tc_val(node_val_mat.T, val_tok_embed.T, pe_mini)
  out = _tc_asm(
      val_out, node_rows.reshape(N_ROWS // 2, DP),
      edge_rows.reshape(N_ROWS // 2, DP), pe_mini, seso)
  return out.reshape(3 * L, D, B).swapaxes(1, 2)


# speedup vs baseline: 1.0056x; 1.0056x over previous
import functools

import jax
import jax.numpy as jnp
import numpy as np
from jax import lax
from jax.experimental import pallas as pl
from jax.experimental.pallas import tpu as pltpu
from jax.experimental.pallas import tpu_sc as plsc

L, B, D = 200, 256, 64
K = 1000
N_ROWS = L * B
DP = 128

_SC_INFO = plsc.get_sparse_core_info()
_NC = _SC_INFO.num_cores
_NS = _SC_INFO.num_subcores
_NW = _NC * _NS
_CHUNK = N_ROWS // _NW
_HALF = _CHUNK // 2
_PIECES = [(o, min(128, _HALF - o)) for o in range(0, _HALF, 128)]

_BL = 50
_NSTEPS = L // _BL
_BLV = 10
_NSTEPS_V = (L + _BLV - 1) // _BLV


def _sc_gather_fn():
  mesh = plsc.VectorSubcoreMesh(core_axis_name="c", subcore_axis_name="s")

  @functools.partial(
      pl.kernel,
      mesh=mesh,
      compiler_params=pltpu.CompilerParams(use_tc_tiling_on_sc=False),
      out_type=(
          jax.ShapeDtypeStruct((N_ROWS, D), jnp.float32),
          jax.ShapeDtypeStruct((N_ROWS, D), jnp.float32),
      ),
      scratch_types=[
          pltpu.VMEM((_CHUNK,), jnp.int32),
          pltpu.VMEM((_HALF, DP), jnp.float32),
          pltpu.VMEM_SHARED((1000, DP), jnp.float32),
          pltpu.SemaphoreType.DMA,
      ],
  )
  def sc_gather(node_idx_h, edge_idx_h, node_tab_h, edge_tab_h,
                node_out_h, edge_out_h, idx_v, rows_v, etab_s, sem):
    wid = lax.axis_index("s") * _NC + lax.axis_index("c")
    base = wid * _CHUNK

    @pl.when(lax.axis_index("s") == 0)
    def _():
      pltpu.sync_copy(edge_tab_h, etab_s)

    def gather_to(tab, src_idx_h, out_h):
      pltpu.sync_copy(src_idx_h.at[pl.ds(base, _CHUNK)], idx_v)
      for half in range(2):
        hoff = half * _HALF
        handles = []
        for off, sz in _PIECES:
          handles.append(
              pltpu.async_copy(
                  tab.at[idx_v.at[pl.ds(hoff + off, sz)]],
                  rows_v.at[pl.ds(off, sz)],
                  sem,
              ))
        for h in handles:
          h.wait()
        pltpu.sync_copy(rows_v.at[:, pl.ds(0, D)],
                        out_h.at[pl.ds(base + hoff, _HALF)])

    gather_to(node_tab_h, node_idx_h, node_out_h)
    plsc.subcore_barrier()
    gather_to(etab_s, edge_idx_h, edge_out_h)

  return sc_gather


_sc_gather = _sc_gather_fn()


def _tc_val_body(vm_ref, vt_ref, pe_ref, out_ref):
  y_t = jax.lax.dot_general(
      vt_ref[...], vm_ref[...], (((1,), (0,)), ((), ())),
      preferred_element_type=jnp.float32)
  for j in range(_BLV):
    out_ref[0, j] = y_t[:, j * B:(j + 1) * B] + pe_ref[j, :, :1]


_tc_val = pl.pallas_call(
    _tc_val_body,
    grid=(_NSTEPS_V,),
    in_specs=[
        pl.BlockSpec((K, _BLV * B), lambda l: (0, l)),
        pl.BlockSpec((D, K), lambda l: (0, 0)),
        pl.BlockSpec((_BLV, D, 8), lambda l: (l, 0, 0)),
    ],
    out_specs=pl.BlockSpec((1, _BLV, D, B), lambda l: (2, l, 0, 0)),
    out_shape=jax.ShapeDtypeStruct((3, L, D, B), jnp.float32),
)


_NPAIR = B // 2


def _tc_asm_body(buf_ref, nv_ref, ev_ref, pe_ref, seso_ref, out_ref):
  del buf_ref
  seso = seso_ref[...]
  for part, ref in ((0, nv_ref), (1, ev_ref)):
    for j in range(_BL):
      pj = ref[pl.ds(j * _NPAIR, _NPAIR), :]
      lhs = jnp.concatenate([pj[:, :D], pj[:, D:]], axis=0)
      xt = jax.lax.dot_general(
          lhs, seso, (((0,), (0,)), ((), ())),
          preferred_element_type=jnp.float32)
      out_ref[part, j] = xt + pe_ref[j, :, :1]


_tc_asm = pl.pallas_call(
    _tc_asm_body,
    grid=(_NSTEPS,),
    in_specs=[
        pl.BlockSpec(memory_space=pltpu.MemorySpace.HBM),
        pl.BlockSpec((_BL * _NPAIR, DP), lambda l: (l, 0)),
        pl.BlockSpec((_BL * _NPAIR, DP), lambda l: (l, 0)),
        pl.BlockSpec((_BL, D, 8), lambda l: (l, 0, 0)),
        pl.BlockSpec((B, B), lambda l: (0, 0)),
    ],
    out_specs=pl.BlockSpec((2, _BL, D, B), lambda l: (0, l, 0, 0)),
    out_shape=jax.ShapeDtypeStruct((3, L, D, B), jnp.float32),
    input_output_aliases={0: 0},
)


def _pad_body(in_ref, out_ref):
  xt = in_ref[...].T
  out_ref[...] = jnp.concatenate([xt, jnp.zeros_like(xt)], axis=1)


def _make_pad(n_rows, block_rows):
  return pl.pallas_call(
      _pad_body,
      grid=((n_rows + block_rows - 1) // block_rows,),
      in_specs=[pl.BlockSpec((D, block_rows), lambda i: (0, i))],
      out_specs=pl.BlockSpec((block_rows, DP), lambda i: (i, 0)),
      out_shape=jax.ShapeDtypeStruct((n_rows, DP), jnp.float32),
  )


_pad_node = _make_pad(100000, 16384)
_pad_edge = _make_pad(1000, 1000)


def _pos_encoding_np():
  pos = np.arange(L, dtype=np.float32)[:, None]
  div = np.exp(np.arange(0, D, 2, dtype=np.float32) * (-np.log(10000.0) / D))
  pe = np.zeros((L, D), dtype=np.float32)
  pe[:, 0::2] = np.sin(pos * div).astype(np.float32)
  pe[:, 1::2] = np.cos(pos * div).astype(np.float32)
  return pe


_PE_MINI = np.broadcast_to(_pos_encoding_np()[:, :, None], (L, D, 8)).copy()
_SESO = np.zeros((B, B), dtype=np.float32)
_SESO[np.arange(B // 2), 2 * np.arange(B // 2)] = 1.0
_SESO[B // 2 + np.arange(B // 2), 2 * np.arange(B // 2) + 1] = 1.0


def kernel(node_idx, edge_idx, node_val_mat, node_embed_table,
           edge_embed_table, val_tok_embed):
  pe_mini = jnp.asarray(_PE_MINI)
  seso = jnp.asarray(_SESO)
  node_tab_p = _pad_node(node_embed_table.T)
  edge_tab_p = _pad_edge(edge_embed_table.T)
  node_rows, edge_rows = _sc_gather(
      node_idx.reshape(-1), edge_idx.reshape(-1), node_tab_p, edge_tab_p)
  val_out = _tc_val(node_val_mat.T, val_tok_embed.T, pe_mini)
  out = _tc_asm(
      val_out, node_rows.reshape(N_ROWS // 2, DP),
      edge_rows.reshape(N_ROWS // 2, DP), pe_mini, seso)
  return out.reshape(3 * L, D, B).swapaxes(1, 2)
